# baseline (speedup 1.0000x reference)
import functools

import jax
import jax.numpy as jnp
from jax import lax
from jax.experimental import pallas as pl
from jax.experimental.pallas import tpu as pltpu
from jax.experimental.pallas import tpu_sc as plsc

VOCAB = 100001
EMBED = 64
BATCH = 1024
CTX = 20

_INFO = plsc.get_sparse_core_info()
_NC = _INFO.num_cores
_NS = _INFO.num_subcores
_NW = _NC * _NS
_BPW = BATCH // _NW
_IPW = _BPW * CTX
_LANES = 16
_DCH = EMBED // _LANES


def _pool_body(ctx_hbm, table_hbm, out_hbm, idx_v, rows_v, acc_v, sem):
    wid = lax.axis_index("s") * _NC + lax.axis_index("c")
    base = wid * _IPW
    pltpu.sync_copy(ctx_hbm.at[pl.ds(base, _IPW)], idx_v)
    pltpu.async_copy(table_hbm.at[idx_v], rows_v, sem).wait()

    def body(r, _):
        row0 = r * CTX
        for c in range(_DCH):
            acc = rows_v[row0, pl.ds(c * _LANES, _LANES)]
            for j in range(1, CTX):
                acc = acc + rows_v[row0 + j, pl.ds(c * _LANES, _LANES)]
            acc_v[pl.ds(r * EMBED + c * _LANES, _LANES)] = acc * (1.0 / CTX)
        return 0

    lax.fori_loop(0, _BPW, body, 0)
    pltpu.sync_copy(acc_v, out_hbm.at[pl.ds(wid * _BPW * EMBED, _BPW * EMBED)])


_pool = functools.partial(
    pl.kernel,
    out_type=jax.ShapeDtypeStruct((BATCH * EMBED,), jnp.float32),
    mesh=plsc.VectorSubcoreMesh(core_axis_name="c", subcore_axis_name="s"),
    scratch_types=[
        pltpu.VMEM((_IPW,), jnp.int32),
        pltpu.VMEM((_IPW, EMBED), jnp.float32),
        pltpu.VMEM((_BPW * EMBED,), jnp.float32),
        pltpu.SemaphoreType.DMA,
    ],
    compiler_params=pltpu.CompilerParams(use_tc_tiling_on_sc=False),
)(_pool_body)


_VT = 2048


def _proj_body(avg_ref, w_ref, b_ref, out_ref):
    out_ref[...] = lax.dot_general(
        avg_ref[...], w_ref[...],
        (((1,), (1,)), ((), ())),
        preferred_element_type=jnp.float32,
    ) + b_ref[...]


def _projection(avg, W, b2):
    grid = (pl.cdiv(VOCAB, _VT),)
    return pl.pallas_call(
        _proj_body,
        grid=grid,
        in_specs=[
            pl.BlockSpec((BATCH, EMBED), lambda i: (0, 0)),
            pl.BlockSpec((_VT, EMBED), lambda i: (i, 0)),
            pl.BlockSpec((1, _VT), lambda i: (0, i)),
        ],
        out_specs=pl.BlockSpec((BATCH, _VT), lambda i: (0, i)),
        out_shape=jax.ShapeDtypeStruct((BATCH, VOCAB), jnp.float32),
    )(avg, W, b2)


def kernel(context, table, W, b):
    ctx_flat = context.reshape(-1)
    avg_flat = _pool(ctx_flat, table)
    avg = avg_flat.reshape(BATCH, EMBED)
    return _projection(avg, W, b.reshape(1, VOCAB))


# speedup vs baseline: 1.0082x; 1.0082x over previous
import functools

import jax
import jax.numpy as jnp
from jax import lax
from jax.experimental import pallas as pl
from jax.experimental.pallas import tpu as pltpu
from jax.experimental.pallas import tpu_sc as plsc

VOCAB = 100001
EMBED = 64
BATCH = 1024
CTX = 20

_INFO = plsc.get_sparse_core_info()
_NC = _INFO.num_cores
_NS = _INFO.num_subcores
_NW = _NC * _NS
_BPW = BATCH // _NW
_IPW = _BPW * CTX
_LANES = 16
_DCH = EMBED // _LANES


def _pool_body(ctx_hbm, table_hbm, out_hbm, idx_v, rows_v, acc_v, sem):
    wid = lax.axis_index("s") * _NC + lax.axis_index("c")
    base = wid * _IPW
    pltpu.sync_copy(ctx_hbm.at[pl.ds(base, _IPW)], idx_v)
    pltpu.async_copy(table_hbm.at[idx_v], rows_v, sem).wait()

    def body(r, _):
        row0 = r * CTX
        for c in range(_DCH):
            acc = rows_v[row0, pl.ds(c * _LANES, _LANES)]
            for j in range(1, CTX):
                acc = acc + rows_v[row0 + j, pl.ds(c * _LANES, _LANES)]
            acc_v[pl.ds(r * EMBED + c * _LANES, _LANES)] = acc * (1.0 / CTX)
        return 0

    lax.fori_loop(0, _BPW, body, 0)
    pltpu.sync_copy(acc_v, out_hbm.at[pl.ds(wid * _BPW * EMBED, _BPW * EMBED)])


_pool = functools.partial(
    pl.kernel,
    out_type=jax.ShapeDtypeStruct((BATCH * EMBED,), jnp.float32),
    mesh=plsc.VectorSubcoreMesh(core_axis_name="c", subcore_axis_name="s"),
    scratch_types=[
        pltpu.VMEM((_IPW,), jnp.int32),
        pltpu.VMEM((_IPW, EMBED), jnp.float32),
        pltpu.VMEM((_BPW * EMBED,), jnp.float32),
        pltpu.SemaphoreType.DMA,
    ],
    compiler_params=pltpu.CompilerParams(use_tc_tiling_on_sc=False),
)(_pool_body)


_VT = 4096


def _proj_body(avg_ref, w_ref, b_ref, out_ref):
    out_ref[...] = lax.dot_general(
        avg_ref[...], w_ref[...],
        (((1,), (1,)), ((), ())),
        preferred_element_type=jnp.float32,
    ) + b_ref[...]


def _projection(avg, W, b2):
    grid = (pl.cdiv(VOCAB, _VT),)
    return pl.pallas_call(
        _proj_body,
        grid=grid,
        in_specs=[
            pl.BlockSpec((BATCH, EMBED), lambda i: (0, 0)),
            pl.BlockSpec((_VT, EMBED), lambda i: (i, 0)),
            pl.BlockSpec((1, _VT), lambda i: (0, i)),
        ],
        out_specs=pl.BlockSpec((BATCH, _VT), lambda i: (0, i)),
        out_shape=jax.ShapeDtypeStruct((BATCH, VOCAB), jnp.float32),
    )(avg, W, b2)


def kernel(context, table, W, b):
    ctx_flat = context.reshape(-1)
    avg_flat = _pool(ctx_flat, table)
    avg = avg_flat.reshape(BATCH, EMBED)
    return _projection(avg, W, b.reshape(1, VOCAB))
